# Initial kernel scaffold; baseline (speedup 1.0000x reference)
import jax
import jax.numpy as jnp
from jax.experimental import pallas as pl


def kernel(token_ids, segment_ids, token_table, pos_table, seg_table, ln_gamma, ln_beta):
    raise NotImplementedError("write your pallas kernel here")


# speedup vs baseline: 2.3905x; 2.3905x over previous
import functools
import math

import jax
import jax.numpy as jnp
from jax import lax
from jax.experimental import pallas as pl
from jax.experimental.pallas import tpu as pltpu
from jax.experimental.pallas import tpu_sc as plsc

_D = 768
_SQRT_D = math.sqrt(_D)
_EPS = 1e-5

_GATHER_W = 64


def _sc_gather(table, flat_idx):
    n = flat_idx.shape[0]
    d = table.shape[1]
    nw = 32
    per_w = n // nw
    c_rows = _GATHER_W
    nch = per_w // c_rows
    mesh = plsc.VectorSubcoreMesh(core_axis_name="c", subcore_axis_name="s")

    @functools.partial(
        pl.kernel,
        out_type=jax.ShapeDtypeStruct((n, d), table.dtype),
        mesh=mesh,
        scratch_types=[
            pltpu.VMEM((per_w,), jnp.int32),
            pltpu.VMEM((c_rows, d), jnp.float32),
            pltpu.SemaphoreType.DMA,
        ],
    )
    def gather_kernel(table_hbm, idx_hbm, out_hbm, idx_v, buf, sem):
        wid = lax.axis_index("s") * 2 + lax.axis_index("c")
        base = wid * per_w
        pltpu.sync_copy(idx_hbm.at[pl.ds(base, per_w)], idx_v)

        @pl.loop(0, nch)
        def _(c):
            pltpu.async_copy(
                table_hbm.at[idx_v.at[pl.ds(c * c_rows, c_rows)]], buf, sem
            ).wait()
            pltpu.sync_copy(buf, out_hbm.at[pl.ds(base + c * c_rows, c_rows)])

    return gather_kernel(table, flat_idx)


def _ln_body(g_ref, seg_ref, pos_ref, segtab_ref, gamma_ref, beta_ref, o_ref):
    x = g_ref[...] * _SQRT_D + pos_ref[...][None, :, :]
    seg_f = seg_ref[...].astype(jnp.float32)[..., None]
    x = x + segtab_ref[0] + seg_f * (segtab_ref[1] - segtab_ref[0])
    mean = jnp.mean(x, axis=-1, keepdims=True)
    xc = x - mean
    var = jnp.mean(xc * xc, axis=-1, keepdims=True)
    xn = xc * lax.rsqrt(var + _EPS)
    o_ref[...] = xn * gamma_ref[...] + beta_ref[...]


def _tc_ln(gathered, segment_ids, pos_table, seg_table, ln_gamma, ln_beta,
           interpret=False):
    b, s = segment_ids.shape
    d = gathered.shape[-1]
    g3 = gathered.reshape(b, s, d)
    bb = 8
    return pl.pallas_call(
        _ln_body,
        grid=(b // bb,),
        in_specs=[
            pl.BlockSpec((bb, s, d), lambda i: (i, 0, 0)),
            pl.BlockSpec((bb, s), lambda i: (i, 0)),
            pl.BlockSpec((s, d), lambda i: (0, 0)),
            pl.BlockSpec((2, d), lambda i: (0, 0)),
            pl.BlockSpec((d,), lambda i: (0,)),
            pl.BlockSpec((d,), lambda i: (0,)),
        ],
        out_specs=pl.BlockSpec((bb, s, d), lambda i: (i, 0, 0)),
        out_shape=jax.ShapeDtypeStruct((b, s, d), gathered.dtype),
        interpret=interpret,
    )(g3, segment_ids, pos_table[:s], seg_table, ln_gamma, ln_beta)


def kernel(token_ids, segment_ids, token_table, pos_table, seg_table,
           ln_gamma, ln_beta):
    b, s = token_ids.shape
    flat_ids = token_ids.reshape(b * s)
    gathered = _sc_gather(token_table, flat_ids)
    return _tc_ln(gathered, segment_ids, pos_table, seg_table,
                  ln_gamma, ln_beta)


# speedup vs baseline: 2.5335x; 1.0598x over previous
import functools
import math

import jax
import jax.numpy as jnp
from jax import lax
from jax.experimental import pallas as pl
from jax.experimental.pallas import tpu as pltpu
from jax.experimental.pallas import tpu_sc as plsc

_D = 768
_SQRT_D = math.sqrt(_D)
_EPS = 1e-5

_GATHER_W = 64


def _sc_gather(table, flat_idx):
    n = flat_idx.shape[0]
    d = table.shape[1]
    nw = 32
    per_w = n // nw
    c_rows = _GATHER_W
    nch = per_w // c_rows
    mesh = plsc.VectorSubcoreMesh(core_axis_name="c", subcore_axis_name="s")

    @functools.partial(
        pl.kernel,
        out_type=jax.ShapeDtypeStruct((n, d), table.dtype),
        mesh=mesh,
        scratch_types=[
            pltpu.VMEM((per_w,), jnp.int32),
            pltpu.VMEM((c_rows, d), jnp.float32),
            pltpu.VMEM((c_rows, d), jnp.float32),
            pltpu.SemaphoreType.DMA,
            pltpu.SemaphoreType.DMA,
            pltpu.SemaphoreType.DMA,
            pltpu.SemaphoreType.DMA,
        ],
    )
    def gather_kernel(table_hbm, idx_hbm, out_hbm, idx_v, buf0, buf1,
                      sg0, sg1, so0, so1):
        wid = lax.axis_index("s") * 2 + lax.axis_index("c")
        base = wid * per_w
        pltpu.sync_copy(idx_hbm.at[pl.ds(base, per_w)], idx_v)

        def g_start(c, buf, sem):
            pltpu.async_copy(
                table_hbm.at[idx_v.at[pl.ds(c * c_rows, c_rows)]], buf, sem
            )

        def g_wait(buf, sem):
            pltpu.make_async_copy(
                table_hbm.at[idx_v.at[pl.ds(0, c_rows)]], buf, sem
            ).wait()

        def o_start(c, buf, sem):
            pltpu.async_copy(buf, out_hbm.at[pl.ds(base + c * c_rows, c_rows)], sem)

        def o_wait(buf, sem):
            pltpu.make_async_copy(buf, out_hbm.at[pl.ds(base, c_rows)], sem).wait()

        g_start(0, buf0, sg0)

        @pl.loop(0, nch, step=2)
        def _(c):
            g_wait(buf0, sg0)

            @pl.when(c >= 2)
            def _():
                o_wait(buf1, so1)

            g_start(c + 1, buf1, sg1)
            o_start(c, buf0, so0)
            g_wait(buf1, sg1)

            @pl.when(c + 2 < nch)
            def _():
                o_wait(buf0, so0)
                g_start(c + 2, buf0, sg0)

            o_start(c + 1, buf1, so1)

        o_wait(buf0, so0)
        o_wait(buf1, so1)

    return gather_kernel(table, flat_idx)


def _ln_body(g_ref, seg_ref, pos_ref, segtab_ref, gamma_ref, beta_ref, o_ref):
    x = g_ref[...] * _SQRT_D + pos_ref[...][None, :, :]
    seg_f = seg_ref[...].astype(jnp.float32)[..., None]
    x = x + segtab_ref[0] + seg_f * (segtab_ref[1] - segtab_ref[0])
    mean = jnp.mean(x, axis=-1, keepdims=True)
    xc = x - mean
    var = jnp.mean(xc * xc, axis=-1, keepdims=True)
    xn = xc * lax.rsqrt(var + _EPS)
    o_ref[...] = xn * gamma_ref[...] + beta_ref[...]


def _tc_ln(gathered, segment_ids, pos_table, seg_table, ln_gamma, ln_beta,
           interpret=False):
    b, s = segment_ids.shape
    d = gathered.shape[-1]
    g3 = gathered.reshape(b, s, d)
    bb = 8
    return pl.pallas_call(
        _ln_body,
        grid=(b // bb,),
        in_specs=[
            pl.BlockSpec((bb, s, d), lambda i: (i, 0, 0)),
            pl.BlockSpec((bb, s), lambda i: (i, 0)),
            pl.BlockSpec((s, d), lambda i: (0, 0)),
            pl.BlockSpec((2, d), lambda i: (0, 0)),
            pl.BlockSpec((d,), lambda i: (0,)),
            pl.BlockSpec((d,), lambda i: (0,)),
        ],
        out_specs=pl.BlockSpec((bb, s, d), lambda i: (i, 0, 0)),
        out_shape=jax.ShapeDtypeStruct((b, s, d), gathered.dtype),
        interpret=interpret,
    )(g3, segment_ids, pos_table[:s], seg_table, ln_gamma, ln_beta)


def kernel(token_ids, segment_ids, token_table, pos_table, seg_table,
           ln_gamma, ln_beta):
    b, s = token_ids.shape
    flat_ids = token_ids.reshape(b * s)
    gathered = _sc_gather(token_table, flat_ids)
    return _tc_ln(gathered, segment_ids, pos_table, seg_table,
                  ln_gamma, ln_beta)


# speedup vs baseline: 2.5641x; 1.0121x over previous
import functools
import math

import jax
import jax.numpy as jnp
from jax import lax
from jax.experimental import pallas as pl
from jax.experimental.pallas import tpu as pltpu
from jax.experimental.pallas import tpu_sc as plsc

_D = 768
_SQRT_D = math.sqrt(_D)
_EPS = 1e-5

_GATHER_W = 64


def _sc_gather(table, flat_idx, c_rows=_GATHER_W):
    n = flat_idx.shape[0]
    d = table.shape[1]
    nw = 32
    per_w = n // nw
    nch = per_w // c_rows
    assert n % nw == 0 and per_w % c_rows == 0 and nch % 2 == 0
    mesh = plsc.VectorSubcoreMesh(core_axis_name="c", subcore_axis_name="s")

    @functools.partial(
        pl.kernel,
        out_type=jax.ShapeDtypeStruct((n, d), table.dtype),
        mesh=mesh,
        scratch_types=[
            pltpu.VMEM((per_w,), jnp.int32),
            pltpu.VMEM((c_rows, d), jnp.float32),
            pltpu.VMEM((c_rows, d), jnp.float32),
            pltpu.SemaphoreType.DMA,
            pltpu.SemaphoreType.DMA,
            pltpu.SemaphoreType.DMA,
            pltpu.SemaphoreType.DMA,
        ],
    )
    def gather_kernel(table_hbm, idx_hbm, out_hbm, idx_v, buf0, buf1,
                      sg0, sg1, so0, so1):
        wid = lax.axis_index("s") * 2 + lax.axis_index("c")
        base = wid * per_w
        pltpu.sync_copy(idx_hbm.at[pl.ds(base, per_w)], idx_v)

        def g_start(c, buf, sem):
            pltpu.async_copy(
                table_hbm.at[idx_v.at[pl.ds(c * c_rows, c_rows)]], buf, sem
            )

        def g_wait(buf, sem):
            pltpu.make_async_copy(
                table_hbm.at[idx_v.at[pl.ds(0, c_rows)]], buf, sem
            ).wait()

        def o_start(c, buf, sem):
            pltpu.async_copy(buf, out_hbm.at[pl.ds(base + c * c_rows, c_rows)], sem)

        def o_wait(buf, sem):
            pltpu.make_async_copy(buf, out_hbm.at[pl.ds(base, c_rows)], sem).wait()

        g_start(0, buf0, sg0)

        @pl.loop(0, nch, step=2)
        def _(c):
            g_wait(buf0, sg0)

            @pl.when(c >= 2)
            def _():
                o_wait(buf1, so1)

            g_start(c + 1, buf1, sg1)
            o_start(c, buf0, so0)
            g_wait(buf1, sg1)

            @pl.when(c + 2 < nch)
            def _():
                o_wait(buf0, so0)
                g_start(c + 2, buf0, sg0)

            o_start(c + 1, buf1, so1)

        o_wait(buf0, so0)
        o_wait(buf1, so1)

    return gather_kernel(table, flat_idx)


def _ln_body(g_ref, seg_ref, pos_ref, segtab_ref, gamma_ref, beta_ref, o_ref):
    x = g_ref[...] * _SQRT_D + pos_ref[...][None, :, :]
    seg_f = seg_ref[...].astype(jnp.float32)[..., None]
    x = x + segtab_ref[0] + seg_f * (segtab_ref[1] - segtab_ref[0])
    mean = jnp.mean(x, axis=-1, keepdims=True)
    xc = x - mean
    var = jnp.mean(xc * xc, axis=-1, keepdims=True)
    xn = xc * lax.rsqrt(var + _EPS)
    o_ref[...] = xn * gamma_ref[...] + beta_ref[...]


def _tc_ln(gathered, segment_ids, pos_table, seg_table, ln_gamma, ln_beta,
           interpret=False):
    b, s = segment_ids.shape
    d = gathered.shape[-1]
    g3 = gathered.reshape(b, s, d)
    bb = 8
    return pl.pallas_call(
        _ln_body,
        grid=(b // bb,),
        in_specs=[
            pl.BlockSpec((bb, s, d), lambda i: (i, 0, 0)),
            pl.BlockSpec((bb, s), lambda i: (i, 0)),
            pl.BlockSpec((s, d), lambda i: (0, 0)),
            pl.BlockSpec((2, d), lambda i: (0, 0)),
            pl.BlockSpec((d,), lambda i: (0,)),
            pl.BlockSpec((d,), lambda i: (0,)),
        ],
        out_specs=pl.BlockSpec((bb, s, d), lambda i: (i, 0, 0)),
        out_shape=jax.ShapeDtypeStruct((b, s, d), gathered.dtype),
        interpret=interpret,
    )(g3, segment_ids, pos_table[:s], seg_table, ln_gamma, ln_beta)


def _ln_body_aliased(g_ref, seg_ref, pos_ref, segtab_ref, gamma_ref, beta_ref,
                     prev_ref, o_ref):
    del prev_ref
    _ln_body(g_ref, seg_ref, pos_ref, segtab_ref, gamma_ref, beta_ref, o_ref)


def _tc_ln_chunk(gathered, seg_k, pos_table, seg_table, ln_gamma, ln_beta,
                 prev_out, k, b):
    bc, s = seg_k.shape
    d = gathered.shape[-1]
    g3 = gathered.reshape(bc, s, d)
    bb = 8
    nblk = bc // bb
    off = k * nblk
    in_specs = [
        pl.BlockSpec((bb, s, d), lambda i: (i, 0, 0)),
        pl.BlockSpec((bb, s), lambda i: (i, 0)),
        pl.BlockSpec((s, d), lambda i: (0, 0)),
        pl.BlockSpec((2, d), lambda i: (0, 0)),
        pl.BlockSpec((d,), lambda i: (0,)),
        pl.BlockSpec((d,), lambda i: (0,)),
    ]
    args = [g3, seg_k, pos_table[:s], seg_table, ln_gamma, ln_beta]
    kwargs = {}
    body = _ln_body
    if prev_out is not None:
        body = _ln_body_aliased
        in_specs.append(pl.BlockSpec((8, 8, 128), lambda i: (0, 0, 0)))
        args.append(prev_out)
        kwargs["input_output_aliases"] = {6: 0}
    return pl.pallas_call(
        body,
        grid=(nblk,),
        in_specs=in_specs,
        out_specs=pl.BlockSpec((bb, s, d), lambda i: (off + i, 0, 0)),
        out_shape=jax.ShapeDtypeStruct((b, s, d), gathered.dtype),
        **kwargs,
    )(*args)


_N_CHUNKS = 4
_CHUNK_GATHER_W = 32


def kernel(token_ids, segment_ids, token_table, pos_table, seg_table,
           ln_gamma, ln_beta):
    b, s = token_ids.shape
    bc = b // _N_CHUNKS
    out = None
    for k in range(_N_CHUNKS):
        ids_k = lax.slice_in_dim(token_ids, k * bc, (k + 1) * bc, axis=0)
        seg_k = lax.slice_in_dim(segment_ids, k * bc, (k + 1) * bc, axis=0)
        gathered = _sc_gather(token_table, ids_k.reshape(bc * s),
                              c_rows=_CHUNK_GATHER_W)
        out = _tc_ln_chunk(gathered, seg_k, pos_table, seg_table,
                           ln_gamma, ln_beta, out, k, b)
    return out
